# Initial kernel scaffold; baseline (speedup 1.0000x reference)
import jax
import jax.numpy as jnp
from jax.experimental import pallas as pl


def kernel(latent_z, edge_idx, W1a, b1a, W1b, b1b, W2a, b2a, W2b, b2b, Wc1, bc1, Wc2, bc2):
    raise NotImplementedError("write your pallas kernel here")


# speedup vs baseline: 1.1632x; 1.1632x over previous
import jax
import jax.numpy as jnp
from jax.experimental import pallas as pl
from jax.experimental.pallas import tpu as pltpu

N = 19
E = 342
LATENT = 512
HID = 2048
T = 4096
KT = N * T


def _adj_times(e_ref, x):
    e = e_ref[...]
    src = e[0:1, :]
    dst = e[1:2, :]
    ii = jax.lax.broadcasted_iota(jnp.int32, (N, E), 0)
    d1h = (ii == dst).astype(jnp.float32)
    s1h = (ii == src).astype(jnp.float32)
    a = jax.lax.dot_general(d1h, s1h, (((1,), (1,)), ((), ())),
                            preferred_element_type=jnp.float32,
                            precision=jax.lax.Precision.HIGHEST)
    return x + jnp.dot(a, x, precision=jax.lax.Precision.HIGHEST)


def _gin_a_body(e_ref, x_ref, w_ref, b_ref, o_ref, agg_ref):
    @pl.when(pl.program_id(0) == 0)
    def _():
        agg_ref[...] = _adj_times(e_ref, x_ref[...])
    o_ref[...] = jnp.maximum(
        jnp.dot(agg_ref[...], w_ref[...], preferred_element_type=jnp.float32)
        + b_ref[...], 0.0)


def _mm_body(x_ref, w_ref, b_ref, o_ref, *, relu):
    r = jnp.dot(x_ref[...], w_ref[...], preferred_element_type=jnp.float32) + b_ref[...]
    o_ref[...] = jnp.maximum(r, 0.0) if relu else r


def _cls_body(f_ref, w_ref, b1_ref, w2_ref, b2_ref, o_ref, acc_ref):
    k = pl.program_id(0)
    nk = pl.num_programs(0)

    @pl.when(k == 0)
    def _():
        acc_ref[...] = jnp.zeros(acc_ref.shape, acc_ref.dtype)

    w = w_ref[...].astype(jnp.bfloat16).astype(jnp.float32)
    f = f_ref[...].astype(jnp.bfloat16).astype(jnp.float32)
    acc_ref[...] += jnp.sum(w * f, axis=0, keepdims=True)

    @pl.when(k == nk - 1)
    def _():
        logits = acc_ref[...] + b1_ref[...]
        o_ref[...] = jax.nn.sigmoid(
            jnp.dot(logits, w2_ref[...], preferred_element_type=jnp.float32)
            + b2_ref[...])


def _full(shape):
    return pl.BlockSpec(shape, lambda j: (0, 0))


def _params():
    return pltpu.CompilerParams(dimension_semantics=("arbitrary",))


def kernel(latent_z, edge_idx, W1a, b1a, W1b, b1b, W2a, b2a, W2b, b2b, Wc1, bc1, Wc2, bc2):
    e = edge_idx.astype(jnp.int32)
    b1a2, b1b2 = b1a.reshape(1, HID), b1b.reshape(1, HID)
    b2a2, b2b2 = b2a.reshape(1, T), b2b.reshape(1, T)
    bc12, bc22 = bc1.reshape(1, LATENT), bc2.reshape(1, 1)

    BJ1 = 512
    h = pl.pallas_call(
        _gin_a_body,
        grid=(HID // BJ1,),
        in_specs=[
            _full((2, E)),
            _full((N, LATENT)),
            pl.BlockSpec((LATENT, BJ1), lambda j: (0, j)),
            pl.BlockSpec((1, BJ1), lambda j: (0, j)),
        ],
        out_specs=pl.BlockSpec((N, BJ1), lambda j: (0, j)),
        out_shape=jax.ShapeDtypeStruct((N, HID), jnp.float32),
        scratch_shapes=[pltpu.VMEM((N, LATENT), jnp.float32)],
        compiler_params=_params(),
    )(e, latent_z, W1a, b1a2)

    BJ2 = 512
    h1 = pl.pallas_call(
        lambda x, w, b, o: _mm_body(x, w, b, o, relu=True),
        grid=(HID // BJ2,),
        in_specs=[
            _full((N, HID)),
            pl.BlockSpec((HID, BJ2), lambda j: (0, j)),
            pl.BlockSpec((1, BJ2), lambda j: (0, j)),
        ],
        out_specs=pl.BlockSpec((N, BJ2), lambda j: (0, j)),
        out_shape=jax.ShapeDtypeStruct((N, HID), jnp.float32),
        compiler_params=_params(),
    )(h, W1b, b1b2)

    BJ3 = 512
    g = pl.pallas_call(
        _gin_a_body,
        grid=(T // BJ3,),
        in_specs=[
            _full((2, E)),
            _full((N, HID)),
            pl.BlockSpec((HID, BJ3), lambda j: (0, j)),
            pl.BlockSpec((1, BJ3), lambda j: (0, j)),
        ],
        out_specs=pl.BlockSpec((N, BJ3), lambda j: (0, j)),
        out_shape=jax.ShapeDtypeStruct((N, T), jnp.float32),
        scratch_shapes=[pltpu.VMEM((N, HID), jnp.float32)],
        compiler_params=_params(),
    )(e, h1, W2a, b2a2)

    BJ4 = 512
    recon = pl.pallas_call(
        lambda x, w, b, o: _mm_body(x, w, b, o, relu=False),
        grid=(T // BJ4,),
        in_specs=[
            _full((N, T)),
            pl.BlockSpec((T, BJ4), lambda j: (0, j)),
            pl.BlockSpec((1, BJ4), lambda j: (0, j)),
        ],
        out_specs=pl.BlockSpec((N, BJ4), lambda j: (0, j)),
        out_shape=jax.ShapeDtypeStruct((N, T), jnp.float32),
        compiler_params=_params(),
    )(g, W2b, b2b2)

    BK = 4864
    flat = recon.reshape(KT, 1)
    pred = pl.pallas_call(
        _cls_body,
        grid=(KT // BK,),
        in_specs=[
            pl.BlockSpec((BK, 1), lambda k: (k, 0)),
            pl.BlockSpec((BK, LATENT), lambda k: (k, 0)),
            _full((1, LATENT)),
            _full((LATENT, 1)),
            _full((1, 1)),
        ],
        out_specs=_full((1, 1)),
        out_shape=jax.ShapeDtypeStruct((1, 1), jnp.float32),
        scratch_shapes=[pltpu.VMEM((1, LATENT), jnp.float32)],
        compiler_params=_params(),
    )(flat, Wc1, bc12, Wc2, bc22)

    return (pred.reshape(1), recon)


# speedup vs baseline: 1.5965x; 1.3725x over previous
import jax
import jax.numpy as jnp
from jax.experimental import pallas as pl
from jax.experimental.pallas import tpu as pltpu

N = 19
E = 342
LATENT = 512
HID = 2048
T = 4096

HI = jax.lax.Precision.HIGHEST

P2 = 1
P3 = P2 + 4
P4 = P3 + 8
P5 = P4 + 16
NSTEP = P5 + N


def _adj(e_ref):
    e = e_ref[...]
    src = e[0:1, :]
    dst = e[1:2, :]
    ii = jax.lax.broadcasted_iota(jnp.int32, (N, E), 0)
    d1h = (ii == dst).astype(jnp.float32)
    s1h = (ii == src).astype(jnp.float32)
    return jax.lax.dot_general(d1h, s1h, (((1,), (1,)), ((), ())),
                               preferred_element_type=jnp.float32,
                               precision=HI)


def _body(e_ref, z_ref, w1a_ref, b1a_ref, w1b_ref, b1b_ref, w2a_ref, b2a_ref,
          w2b_ref, b2b_ref, wc1_ref, bc1_ref, wc2_ref, bc2_ref,
          recon_ref, pred_ref,
          h_ref, x2_ref, g_ref, acc1_ref, acc2_ref, acc3_ref, accl_ref):
    s = pl.program_id(0)

    @pl.when(s == 0)
    def _p1():
        a = _adj(e_ref)
        z = z_ref[...]
        x1 = z + jnp.dot(a, z, precision=HI)
        h = jnp.maximum(jnp.dot(x1, w1a_ref[...]) + b1a_ref[...], 0.0)
        for i in range(4):
            h_ref[i] = h[:, i * 512:(i + 1) * 512]

    @pl.when((s >= P2) & (s < P3))
    def _p2():
        k = s - P2
        part = jnp.dot(h_ref[pl.ds(k, 1)][0], w1b_ref[...])

        @pl.when(k == 0)
        def _():
            acc1_ref[...] = part

        @pl.when(k > 0)
        def _():
            acc1_ref[...] += part

    @pl.when(s == P3)
    def _p3a():
        h1 = jnp.maximum(acc1_ref[...] + b1b_ref[...], 0.0)
        a = _adj(e_ref)
        x2 = h1 + jnp.dot(a, h1, precision=HI)
        for i in range(8):
            x2_ref[i] = x2[:, i * 256:(i + 1) * 256]

    @pl.when((s >= P3) & (s < P4))
    def _p3():
        k = s - P3
        part = jnp.dot(x2_ref[pl.ds(k, 1)][0], w2a_ref[...])

        @pl.when(k == 0)
        def _():
            acc2_ref[...] = part

        @pl.when(k > 0)
        def _():
            acc2_ref[...] += part

    @pl.when(s == P4)
    def _p4a():
        g = jnp.maximum(acc2_ref[...] + b2a_ref[...], 0.0)
        for i in range(16):
            g_ref[i] = g[:, i * 256:(i + 1) * 256]

    @pl.when((s >= P4) & (s < P5))
    def _p4():
        k = s - P4
        part = jnp.dot(g_ref[pl.ds(k, 1)][0], w2b_ref[...])

        @pl.when(k == 0)
        def _():
            acc3_ref[...] = part

        @pl.when(k > 0)
        def _():
            acc3_ref[...] += part

        @pl.when(k == 15)
        def _():
            acc3_ref[...] += b2b_ref[...]
            recon_ref[...] = acc3_ref[...]

    @pl.when(s >= P5)
    def _p5():
        n = s - P5
        sel = (jax.lax.broadcasted_iota(jnp.int32, (1, N), 1)
               == n).astype(jnp.float32)
        row = jnp.dot(sel, acc3_ref[...], precision=HI)
        part = jnp.dot(row, wc1_ref[...])

        @pl.when(n == 0)
        def _():
            accl_ref[...] = part

        @pl.when(n > 0)
        def _():
            accl_ref[...] += part

        @pl.when(n == N - 1)
        def _():
            logits = accl_ref[...] + bc1_ref[...]
            pred_ref[...] = jax.nn.sigmoid(
                jnp.dot(logits, wc2_ref[...]) + bc2_ref[...])


def kernel(latent_z, edge_idx, W1a, b1a, W1b, b1b, W2a, b2a, W2b, b2b, Wc1, bc1, Wc2, bc2):
    e = edge_idx.astype(jnp.int32)

    def pin(shape):
        return pl.BlockSpec(shape, lambda s: (0, 0))

    in_specs = [
        pin((2, E)),
        pin((N, LATENT)),
        pin((LATENT, HID)),
        pin((1, HID)),
        pl.BlockSpec((512, HID), lambda s: (jnp.clip(s - P2, 0, 3), 0)),
        pin((1, HID)),
        pl.BlockSpec((256, T), lambda s: (jnp.clip(s - P3, 0, 7), 0)),
        pin((1, T)),
        pl.BlockSpec((256, T), lambda s: (jnp.clip(s - P4, 0, 15), 0)),
        pin((1, T)),
        pl.BlockSpec((T, LATENT), lambda s: (jnp.clip(s - P5, 0, N - 1), 0)),
        pin((1, LATENT)),
        pin((LATENT, 1)),
        pin((1, 1)),
    ]
    out_specs = [pin((N, T)), pin((1, 1))]
    out_shape = [jax.ShapeDtypeStruct((N, T), jnp.float32),
                 jax.ShapeDtypeStruct((1, 1), jnp.float32)]
    scratch = [
        pltpu.VMEM((4, N, 512), jnp.float32),
        pltpu.VMEM((8, N, 256), jnp.float32),
        pltpu.VMEM((16, N, 256), jnp.float32),
        pltpu.VMEM((N, HID), jnp.float32),
        pltpu.VMEM((N, T), jnp.float32),
        pltpu.VMEM((N, T), jnp.float32),
        pltpu.VMEM((1, LATENT), jnp.float32),
    ]

    recon, pred = pl.pallas_call(
        _body,
        grid=(NSTEP,),
        in_specs=in_specs,
        out_specs=out_specs,
        out_shape=out_shape,
        scratch_shapes=scratch,
        compiler_params=pltpu.CompilerParams(
            dimension_semantics=("arbitrary",)),
    )(e, latent_z, W1a, b1a.reshape(1, HID), W1b, b1b.reshape(1, HID),
      W2a, b2a.reshape(1, T), W2b, b2b.reshape(1, T),
      Wc1, bc1.reshape(1, LATENT), Wc2, bc2.reshape(1, 1))

    return (pred.reshape(1), recon)


# speedup vs baseline: 1.6390x; 1.0266x over previous
import jax
import jax.numpy as jnp
from jax.experimental import pallas as pl
from jax.experimental.pallas import tpu as pltpu

N = 19
E = 342
LATENT = 512
HID = 2048
T = 4096

HI = jax.lax.Precision.HIGHEST

P2 = 1
P3 = P2 + 4
P4 = P3 + 8
P5 = P4 + 8
NSTEP = P5 + N


def _adj(e_ref):
    e = e_ref[...]
    src = e[0:1, :]
    dst = e[1:2, :]
    ii = jax.lax.broadcasted_iota(jnp.int32, (N, E), 0)
    d1h = (ii == dst).astype(jnp.float32)
    s1h = (ii == src).astype(jnp.float32)
    return jax.lax.dot_general(d1h, s1h, (((1,), (1,)), ((), ())),
                               preferred_element_type=jnp.float32,
                               precision=HI)


def _body(e_ref, z_ref, w1a_ref, b1a_ref, w1b_ref, b1b_ref, w2a_ref, b2a_ref,
          w2b_ref, b2b_ref, wc1_ref, bc1_ref, wc2_ref, bc2_ref,
          recon_ref, pred_ref,
          h_ref, x2_ref, g_ref, acc1_ref, acc2_ref, acc3_ref, accl_ref):
    s = pl.program_id(0)

    @pl.when(s == 0)
    def _p1():
        a = _adj(e_ref)
        z = z_ref[...]
        x1 = z + jnp.dot(a, z, precision=HI)
        h = jnp.maximum(jnp.dot(x1, w1a_ref[...]) + b1a_ref[...], 0.0)
        for i in range(4):
            h_ref[i] = h[:, i * 512:(i + 1) * 512]

    @pl.when((s >= P2) & (s < P3))
    def _p2():
        k = s - P2
        part = jnp.dot(h_ref[pl.ds(k, 1)][0], w1b_ref[...])

        @pl.when(k == 0)
        def _():
            acc1_ref[...] = part

        @pl.when(k > 0)
        def _():
            acc1_ref[...] += part

    @pl.when(s == P3)
    def _p3a():
        h1 = jnp.maximum(acc1_ref[...] + b1b_ref[...], 0.0)
        a = _adj(e_ref)
        x2 = h1 + jnp.dot(a, h1, precision=HI)
        for i in range(8):
            x2_ref[i] = x2[:, i * 256:(i + 1) * 256]

    @pl.when((s >= P3) & (s < P4))
    def _p3():
        k = s - P3
        part = jnp.dot(x2_ref[pl.ds(k, 1)][0], w2a_ref[...])

        @pl.when(k == 0)
        def _():
            acc2_ref[...] = part

        @pl.when(k > 0)
        def _():
            acc2_ref[...] += part

    @pl.when(s == P4)
    def _p4a():
        g = jnp.maximum(acc2_ref[...] + b2a_ref[...], 0.0)
        for i in range(8):
            g_ref[i] = g[:, i * 512:(i + 1) * 512]

    @pl.when((s >= P4) & (s < P5))
    def _p4():
        k = s - P4
        part = jnp.dot(g_ref[pl.ds(k, 1)][0], w2b_ref[...])

        @pl.when(k == 0)
        def _():
            acc3_ref[...] = part

        @pl.when(k > 0)
        def _():
            acc3_ref[...] += part

        @pl.when(k == 7)
        def _():
            acc3_ref[...] += b2b_ref[...]
            recon_ref[...] = acc3_ref[...]

    @pl.when(s >= P5)
    def _p5():
        n = s - P5
        sel = (jax.lax.broadcasted_iota(jnp.int32, (1, N), 1)
               == n).astype(jnp.float32)
        row = jnp.dot(sel, acc3_ref[...], precision=HI)
        part = jnp.dot(row, wc1_ref[...])

        @pl.when(n == 0)
        def _():
            accl_ref[...] = part

        @pl.when(n > 0)
        def _():
            accl_ref[...] += part

        @pl.when(n == N - 1)
        def _():
            logits = accl_ref[...] + bc1_ref[...]
            pred_ref[...] = jax.nn.sigmoid(
                jnp.dot(logits, wc2_ref[...]) + bc2_ref[...])


def kernel(latent_z, edge_idx, W1a, b1a, W1b, b1b, W2a, b2a, W2b, b2b, Wc1, bc1, Wc2, bc2):
    e = edge_idx.astype(jnp.int32)

    def pin(shape):
        return pl.BlockSpec(shape, lambda s: (0, 0))

    in_specs = [
        pin((2, E)),
        pin((N, LATENT)),
        pin((LATENT, HID)),
        pin((1, HID)),
        pl.BlockSpec((512, HID), lambda s: (jnp.clip(s - P2, 0, 3), 0)),
        pin((1, HID)),
        pl.BlockSpec((256, T), lambda s: (jnp.clip(s - P3, 0, 7), 0)),
        pin((1, T)),
        pl.BlockSpec((512, T), lambda s: (jnp.clip(s - P4, 0, 7), 0)),
        pin((1, T)),
        pl.BlockSpec((T, LATENT), lambda s: (jnp.clip(s - P5, 0, N - 1), 0)),
        pin((1, LATENT)),
        pin((LATENT, 1)),
        pin((1, 1)),
    ]
    out_specs = [pin((N, T)), pin((1, 1))]
    out_shape = [jax.ShapeDtypeStruct((N, T), jnp.float32),
                 jax.ShapeDtypeStruct((1, 1), jnp.float32)]
    scratch = [
        pltpu.VMEM((4, N, 512), jnp.float32),
        pltpu.VMEM((8, N, 256), jnp.float32),
        pltpu.VMEM((8, N, 512), jnp.float32),
        pltpu.VMEM((N, HID), jnp.float32),
        pltpu.VMEM((N, T), jnp.float32),
        pltpu.VMEM((N, T), jnp.float32),
        pltpu.VMEM((1, LATENT), jnp.float32),
    ]

    recon, pred = pl.pallas_call(
        _body,
        grid=(NSTEP,),
        in_specs=in_specs,
        out_specs=out_specs,
        out_shape=out_shape,
        scratch_shapes=scratch,
        compiler_params=pltpu.CompilerParams(
            dimension_semantics=("arbitrary",)),
    )(e, latent_z, W1a, b1a.reshape(1, HID), W1b, b1b.reshape(1, HID),
      W2a, b2a.reshape(1, T), W2b, b2b.reshape(1, T),
      Wc1, bc1.reshape(1, LATENT), Wc2, bc2.reshape(1, 1))

    return (pred.reshape(1), recon)
